# baseline (speedup 1.0000x reference)
import jax
import jax.numpy as jnp
from jax import lax
from jax.experimental import pallas as pl
from jax.experimental.pallas import tpu as pltpu
from jax.experimental.pallas import tpu_sc as plsc

BATCH = 4096
N_TOKENS = 50
TOKENS = BATCH * N_TOKENS
CHUNK = 128
NCHUNKS = TOKENS // CHUNK
NW = 32
CPW = NCHUNKS // NW
D_OUT = 256
N_ROWS = 100000
N_FIELDS = 8
LANES = 16

FIELDS = (
    (0, 0, 64, 0),
    (1, 64, 32, 1),
    (2, 96, 32, 1),
    (3, 128, 32, 1),
    (4, 160, 32, 1),
    (5, 192, 16, 2),
    (6, 208, 16, 3),
    (7, 224, 32, 4),
)
TABLE_WIDTHS = (64, 32, 16, 16, 32)


def _body(idx_hbm, sp_hbm, mv_hbm, ab_hbm, it_hbm, lm_hbm, out_hbm,
          idxr, idxv, bufs0, bufs1, gsem0, gsem1, wsem0, wsem1, isem):
    tables = (sp_hbm, mv_hbm, ab_hbm, it_hbm, lm_hbm)
    bufs = (bufs0, bufs1)
    gsems = (gsem0, gsem1)
    wsems = (wsem0, wsem1)
    wid = lax.axis_index("s") * 2 + lax.axis_index("c")
    chunk0 = wid * CPW
    lanes8 = lax.iota(jnp.int32, LANES) * N_FIELDS

    def deinterleave(b):
        for f in range(N_FIELDS):
            for grp in range(CHUNK // LANES):
                vec = plsc.load_gather(
                    idxr.at[b], [lanes8 + (grp * LANES * N_FIELDS + f)])
                idxv[b, f, pl.ds(grp * LANES, LANES)] = vec

    def fire_gathers(b):
        for i, (row, _, _, targ) in enumerate(FIELDS):
            pltpu.async_copy(tables[targ].at[idxv.at[b, row]], bufs[b][i],
                             gsems[b])

    def wait_gathers(b):
        for i, (row, _, _, targ) in enumerate(FIELDS):
            pltpu.make_async_copy(tables[targ].at[idxv.at[b, row]], bufs[b][i],
                                  gsems[b]).wait()

    def fire_writes(b, g):
        tok = pl.multiple_of(g * CHUNK, CHUNK)
        for i, (_, col, w, _) in enumerate(FIELDS):
            pltpu.async_copy(
                bufs[b][i], out_hbm.at[pl.ds(tok, CHUNK), pl.ds(col, w)],
                wsems[b])

    def wait_writes(b):
        for i, (_, col, w, _) in enumerate(FIELDS):
            pltpu.make_async_copy(
                bufs[b][i], out_hbm.at[pl.ds(0, CHUNK), pl.ds(col, w)],
                wsems[b]).wait()

    pltpu.sync_copy(idx_hbm.at[chunk0], idxr.at[0])
    deinterleave(0)
    fire_gathers(0)

    def pair(jj, carry):
        for b in (0, 1):
            j = jj * 2 + b
            g = chunk0 + j

            @pl.when(j + 1 < CPW)
            def _prefetch_idx():
                pltpu.async_copy(idx_hbm.at[g + 1], idxr.at[1 - b], isem)

            @pl.when(j >= 1)
            def _():
                wait_writes(1 - b)

            @pl.when(j + 1 < CPW)
            def _next_gathers():
                pltpu.make_async_copy(idx_hbm.at[g + 1], idxr.at[1 - b],
                                      isem).wait()
                deinterleave(1 - b)
                fire_gathers(1 - b)

            wait_gathers(b)
            fire_writes(b, g)

        return carry

    lax.fori_loop(0, CPW // 2, pair, 0)
    wait_writes((CPW - 1) % 2)


@jax.jit
def _run(idx, sp, mv, ab, it, lm):
    def field_bufs():
        return tuple(pltpu.VMEM((CHUNK, w), jnp.float32) for _, _, w, _ in FIELDS)
    scratch = [
        pltpu.VMEM((2, CHUNK * N_FIELDS), jnp.int32),
        pltpu.VMEM((2, N_FIELDS, CHUNK), jnp.int32),
        field_bufs(),
        field_bufs(),
        pltpu.SemaphoreType.DMA,
        pltpu.SemaphoreType.DMA,
        pltpu.SemaphoreType.DMA,
        pltpu.SemaphoreType.DMA,
        pltpu.SemaphoreType.DMA,
    ]
    kern = pl.kernel(
        _body,
        out_type=jax.ShapeDtypeStruct((TOKENS, D_OUT), jnp.float32),
        mesh=plsc.VectorSubcoreMesh(core_axis_name="c", subcore_axis_name="s"),
        scratch_types=scratch,
        compiler_params=pltpu.CompilerParams(use_tc_tiling_on_sc=False,
                                             needs_layout_passes=False),
    )
    return kern(idx, sp, mv, ab, it, lm)


def kernel(int_ids, species_emb, move_emb, ability_emb, item_emb, last_move_emb):
    ids = int_ids.astype(jnp.int32)
    pieces = [ids.reshape(-1)]
    for t in (species_emb, move_emb, ability_emb, item_emb, last_move_emb):
        pieces.append(lax.bitcast_convert_type(t, jnp.int32).reshape(-1))
    flat = jnp.concatenate(pieces)

    idx = lax.slice(flat, (0,), (NCHUNKS * CHUNK * N_FIELDS,)).reshape(
        NCHUNKS, CHUNK * N_FIELDS)
    off = NCHUNKS * CHUNK * N_FIELDS
    views = []
    for w in TABLE_WIDTHS:
        v = lax.slice(flat, (off,), (off + N_ROWS * w,)).reshape(N_ROWS, w)
        views.append(lax.bitcast_convert_type(v, jnp.float32))
        off += N_ROWS * w

    out = _run(idx, *views)
    return out.reshape(BATCH, N_TOKENS, D_OUT)
